# baseline (speedup 1.0000x reference)
import functools

import jax
import jax.numpy as jnp
from jax import lax
from jax.experimental import pallas as pl
from jax.experimental.pallas import tpu as pltpu
from jax.experimental.pallas import tpu_sc as plsc

N = 50000
E = 625000
D = 128
H = 128
EPS = 1e-5

NCHUNK = 4
CW = 32
NTILE = 16
BLK = 1280
NBLK = 31
E_PAD = NTILE * NBLK * BLK
PAD_ROWS = 304
ACC_N = N + PAD_ROWS
ZROWS = ACC_N // NTILE
OROWS = 3128
CNTW = 8
KB = BLK // 128


def _seg_body(h4, src2, dst2, zeros32, s_out, gi2, dsw2, rows, acc_sh, sems):
  c = lax.axis_index("c")
  s = lax.axis_index("s")
  G = NBLK * KB

  for l in range(2):
    cc = 2 * c + l
    pltpu.sync_copy(zeros32, acc_sh.at[pl.ds(ZROWS * s, ZROWS)])
    plsc.subcore_barrier()

    def fire_idx(j, p):
      row0 = (s * NBLK + j) * KB
      pltpu.async_copy(src2.at[pl.ds(row0, KB)],
                       gi2.at[pl.ds(p * KB, KB)], sems.at[p])
      pltpu.async_copy(dst2.at[pl.ds(row0, KB)],
                       dsw2.at[pl.ds(p * KB, KB)], sems.at[p])

    def wait_idx(j, p):
      row0 = (s * NBLK + j) * KB
      pltpu.make_async_copy(src2.at[pl.ds(row0, KB)],
                            gi2.at[pl.ds(p * KB, KB)], sems.at[p]).wait()
      pltpu.make_async_copy(dst2.at[pl.ds(row0, KB)],
                            dsw2.at[pl.ds(p * KB, KB)], sems.at[p]).wait()

    def transform(p):
      def outer(k, _):
        def inner(i, _):
          gi2[p * KB + k, pl.ds(i * 16, 16)] = (
              gi2[p * KB + k, pl.ds(i * 16, 16)] * NCHUNK + cc)
          return None
        lax.fori_loop(0, 8, inner, None)
        return None
      lax.fori_loop(0, KB, outer, None)

    def fire_gather(t):
      p = (t // KB) % 2
      rb = t % 3
      pltpu.async_copy(h4.at[gi2.at[p * KB + t % KB]],
                       rows.at[pl.ds(rb * 128, 128)], sems.at[2 + rb])

    def wait_gather(t):
      rb = t % 3
      pltpu.make_async_copy(h4.at[pl.ds(0, 128)],
                            rows.at[pl.ds(rb * 128, 128)],
                            sems.at[2 + rb]).wait()

    def fire_scatter(t):
      p = (t // KB) % 2
      rb = t % 3
      pltpu.async_copy(rows.at[pl.ds(rb * 128, 128)],
                       acc_sh.at[dsw2.at[p * KB + t % KB]],
                       sems.at[5 + rb], add=True)

    def wait_scatter(t):
      rb = t % 3
      pltpu.make_async_copy(h4.at[pl.ds(0, 128)],
                            rows.at[pl.ds(rb * 128, 128)],
                            sems.at[5 + rb]).wait()

    fire_idx(0, 0)
    wait_idx(0, 0)
    transform(0)
    fire_idx(1, 1)
    fire_gather(0)
    fire_gather(1)

    def body(t, _):
      t2 = t + 2

      @pl.when((t2 % KB == 0) & (t2 < G))
      def _():
        wait_idx(t2 // KB, (t2 // KB) % 2)
        transform((t2 // KB) % 2)

      wait_gather(t)
      fire_scatter(t)

      @pl.when(t2 < G)
      def _():
        @pl.when(t >= 1)
        def _():
          wait_scatter(t2)

        fire_gather(t2)

      @pl.when((t % KB == 2) & (t // KB >= 1) & (t // KB + 1 < NBLK))
      def _():
        fire_idx(t // KB + 1, (t // KB + 1) % 2)
      return None

    lax.fori_loop(0, G, body, None)
    wait_scatter(G - 3)
    wait_scatter(G - 2)
    wait_scatter(G - 1)
    plsc.subcore_barrier()

    last = N - 15 * OROWS

    @pl.when(s < 15)
    def _():
      pltpu.sync_copy(acc_sh.at[pl.ds(OROWS * s, OROWS)],
                      s_out.at[cc, pl.ds(OROWS * s, OROWS)])

    @pl.when(s == 15)
    def _():
      pltpu.sync_copy(acc_sh.at[pl.ds(OROWS * 15, last)],
                      s_out.at[cc, pl.ds(OROWS * 15, last)])
    plsc.subcore_barrier()


_seg = functools.partial(
    pl.kernel,
    mesh=plsc.VectorSubcoreMesh(core_axis_name="c", subcore_axis_name="s"),
    compiler_params=pltpu.CompilerParams(use_tc_tiling_on_sc=False),
    out_type=jax.ShapeDtypeStruct((NCHUNK, N, CW), jnp.float32),
    scratch_types=[
        pltpu.VMEM((2 * KB, 128), jnp.int32),
        pltpu.VMEM((2 * KB, 128), jnp.int32),
        pltpu.VMEM((384, CW), jnp.float32),
        pltpu.VMEM_SHARED((ACC_N, CW), jnp.float32),
        pltpu.SemaphoreType.DMA((8,)),
    ],
)(_seg_body)


def _cnt_body(dst2_ui, dst2_iu, zeros8, ones8, cui_out, ciu_out,
              dstw, ones_v, cnt_sh):
  c = lax.axis_index("c")
  s = lax.axis_index("s")

  pltpu.sync_copy(ones8, ones_v)
  pltpu.sync_copy(zeros8, cnt_sh.at[pl.ds(ZROWS * s, ZROWS)])
  plsc.subcore_barrier()

  def count(dst2):
    def block(j, _):
      row0 = (s * NBLK + j) * KB
      pltpu.sync_copy(dst2.at[pl.ds(row0, KB)], dstw)
      for k in range(KB):
        pltpu.sync_copy(ones_v, cnt_sh.at[dstw.at[k]], add=True)
      return None
    lax.fori_loop(0, NBLK, block, None)

  def copy_out(c_out):
    last = N - 15 * OROWS

    @pl.when(s < 15)
    def _():
      pltpu.sync_copy(cnt_sh.at[pl.ds(OROWS * s, OROWS)],
                      c_out.at[pl.ds(OROWS * s, OROWS)])

    @pl.when(s == 15)
    def _():
      pltpu.sync_copy(cnt_sh.at[pl.ds(OROWS * 15, last)],
                      c_out.at[pl.ds(OROWS * 15, last)])

  @pl.when(c == 0)
  def _():
    count(dst2_ui)
  @pl.when(c == 1)
  def _():
    count(dst2_iu)
  plsc.subcore_barrier()

  @pl.when(c == 0)
  def _():
    copy_out(cui_out)
  @pl.when(c == 1)
  def _():
    copy_out(ciu_out)


_cnt = functools.partial(
    pl.kernel,
    mesh=plsc.VectorSubcoreMesh(core_axis_name="c", subcore_axis_name="s"),
    compiler_params=pltpu.CompilerParams(use_tc_tiling_on_sc=False),
    out_type=[
        jax.ShapeDtypeStruct((N, CNTW), jnp.float32),
        jax.ShapeDtypeStruct((N, CNTW), jnp.float32),
    ],
    scratch_types=[
        pltpu.VMEM((KB, 128), jnp.int32),
        pltpu.VMEM((128, CNTW), jnp.float32),
        pltpu.VMEM_SHARED((ACC_N, CNTW), jnp.float32),
    ],
)(_cnt_body)


def _mm(a, b):
  return jnp.dot(a, b, preferred_element_type=jnp.float32,
                 precision=lax.Precision.HIGHEST)


def _proj_body(xu_ref, xi_ref, wu_ref, bu_ref, wi_ref, bi_ref,
               ou_ref, oi_ref):
  ou_ref[...] = _mm(xu_ref[...], wu_ref[...]) + bu_ref[...]
  oi_ref[...] = _mm(xi_ref[...], wi_ref[...]) + bi_ref[...]


_proj = pl.pallas_call(
    _proj_body,
    grid=(50,),
    in_specs=[
        pl.BlockSpec((1000, D), lambda i: (i, 0)),
        pl.BlockSpec((1000, D), lambda i: (i, 0)),
        pl.BlockSpec((D, H), lambda i: (0, 0)),
        pl.BlockSpec((1, H), lambda i: (0, 0)),
        pl.BlockSpec((D, H), lambda i: (0, 0)),
        pl.BlockSpec((1, H), lambda i: (0, 0)),
    ],
    out_specs=[
        pl.BlockSpec((1000, H), lambda i: (i, 0)),
        pl.BlockSpec((1000, H), lambda i: (i, 0)),
    ],
    out_shape=[
        jax.ShapeDtypeStruct((N, H), jnp.float32),
        jax.ShapeDtypeStruct((N, H), jnp.float32),
    ],
)


def _comb_one(i, s_ref, c_ref, h_ref, wl_ref, wr_ref, bl_ref, o_ref, st_ref):
  sv = s_ref[...]
  t = _mm(sv[0], wl_ref[0])
  for k in range(1, NCHUNK):
    t += _mm(sv[k], wl_ref[k])
  cinv = 1.0 / jnp.maximum(c_ref[...][:, :1], 1.0)
  out = t * cinv + _mm(h_ref[...], wr_ref[...]) + bl_ref[...]
  o_ref[...] = out

  @pl.when(i == 0)
  def _():
    st_ref[...] = jnp.zeros_like(st_ref)

  st_ref[0:1, :] += jnp.sum(out, axis=0, keepdims=True)
  st_ref[1:2, :] += jnp.sum(out * out, axis=0, keepdims=True)


def _comb_body(si_ref, ci_ref, hi_ref, wli_ref, wri_ref, bli_ref,
               su_ref, cu_ref, hu_ref, wlu_ref, wru_ref, blu_ref,
               oi_ref, sti_ref, ou_ref, stu_ref):
  i = pl.program_id(0)
  _comb_one(i, si_ref, ci_ref, hi_ref, wli_ref, wri_ref, bli_ref,
            oi_ref, sti_ref)
  _comb_one(i, su_ref, cu_ref, hu_ref, wlu_ref, wru_ref, blu_ref,
            ou_ref, stu_ref)


def _comb_specs():
  return [
      pl.BlockSpec((NCHUNK, 1000, CW), lambda i: (0, i, 0)),
      pl.BlockSpec((1000, CNTW), lambda i: (i, 0)),
      pl.BlockSpec((1000, D), lambda i: (i, 0)),
      pl.BlockSpec((NCHUNK, CW, H), lambda i: (0, 0, 0)),
      pl.BlockSpec((D, H), lambda i: (0, 0)),
      pl.BlockSpec((1, H), lambda i: (0, 0)),
  ]


_comb = pl.pallas_call(
    _comb_body,
    grid=(50,),
    in_specs=_comb_specs() + _comb_specs(),
    out_specs=[
        pl.BlockSpec((1000, H), lambda i: (i, 0)),
        pl.BlockSpec((8, H), lambda i: (0, 0)),
        pl.BlockSpec((1000, H), lambda i: (i, 0)),
        pl.BlockSpec((8, H), lambda i: (0, 0)),
    ],
    out_shape=[
        jax.ShapeDtypeStruct((N, H), jnp.float32),
        jax.ShapeDtypeStruct((8, H), jnp.float32),
        jax.ShapeDtypeStruct((N, H), jnp.float32),
        jax.ShapeDtypeStruct((8, H), jnp.float32),
    ],
)


def _bn_body(xu_ref, scu_ref, shu_ref, xi_ref, sci_ref, shi_ref,
             ou_ref, oi_ref):
  ou_ref[...] = jnp.maximum(xu_ref[...] * scu_ref[...] + shu_ref[...], 0.0)
  oi_ref[...] = jnp.maximum(xi_ref[...] * sci_ref[...] + shi_ref[...], 0.0)


_bn = pl.pallas_call(
    _bn_body,
    grid=(50,),
    in_specs=[
        pl.BlockSpec((1000, H), lambda i: (i, 0)),
        pl.BlockSpec((1, H), lambda i: (0, 0)),
        pl.BlockSpec((1, H), lambda i: (0, 0)),
        pl.BlockSpec((1000, H), lambda i: (i, 0)),
        pl.BlockSpec((1, H), lambda i: (0, 0)),
        pl.BlockSpec((1, H), lambda i: (0, 0)),
    ],
    out_specs=[
        pl.BlockSpec((1000, H), lambda i: (i, 0)),
        pl.BlockSpec((1000, H), lambda i: (i, 0)),
    ],
    out_shape=[
        jax.ShapeDtypeStruct((N, H), jnp.float32),
        jax.ShapeDtypeStruct((N, H), jnp.float32),
    ],
)


def _pad_edges(edge):
  npad = E_PAD - E
  ar = jnp.arange(npad, dtype=jnp.int32)
  pad_src = (ar * 37) % N
  pad_dst = N + ar % PAD_ROWS
  src2 = jnp.concatenate([edge[0], pad_src]).reshape(-1, 128)
  dst2 = jnp.concatenate([edge[1], pad_dst]).reshape(-1, 128)
  return src2, dst2


def _bn_coeffs(st, g, be):
  mu = st[0] / N
  var = jnp.maximum(st[1] / N - mu * mu, 0.0)
  scale = g / jnp.sqrt(var + EPS)
  shift = be - mu * scale
  return scale.reshape(1, H), shift.reshape(1, H)


def kernel(x_user, x_item, edge_ui, edge_iu,
           W_in_user, b_in_user, W_in_item, b_in_item,
           Wl0_ui, bl0_ui, Wr0_ui, Wl0_iu, bl0_iu, Wr0_iu,
           g0_user, be0_user, g0_item, be0_item,
           Wl1_ui, bl1_ui, Wr1_ui, Wl1_iu, bl1_iu, Wr1_iu,
           g1_user, be1_user, g1_item, be1_item):
  src_ui, dst_ui = _pad_edges(edge_ui)
  src_iu, dst_iu = _pad_edges(edge_iu)
  zeros32 = jnp.zeros((ZROWS, CW), jnp.float32)
  zeros8 = jnp.zeros((ZROWS, CNTW), jnp.float32)
  ones8 = jnp.ones((128, CNTW), jnp.float32)

  c_i, c_u = _cnt(dst_ui, dst_iu, zeros8, ones8)

  h_user, h_item = _proj(x_user, x_item,
                         W_in_user, b_in_user.reshape(1, H),
                         W_in_item, b_in_item.reshape(1, H))

  params = [
      (Wl0_ui, bl0_ui, Wr0_ui, Wl0_iu, bl0_iu, Wr0_iu,
       g0_user, be0_user, g0_item, be0_item),
      (Wl1_ui, bl1_ui, Wr1_ui, Wl1_iu, bl1_iu, Wr1_iu,
       g1_user, be1_user, g1_item, be1_item),
  ]
  for (Wl_ui, bl_ui, Wr_ui, Wl_iu, bl_iu, Wr_iu,
       g_u, be_u, g_i, be_i) in params:
    s_i = _seg(h_user.reshape(N * NCHUNK, CW), src_ui, dst_ui, zeros32)
    s_u = _seg(h_item.reshape(N * NCHUNK, CW), src_iu, dst_iu, zeros32)
    out_i, st_i, out_u, st_u = _comb(
        s_i, c_i, h_item, Wl_ui.reshape(NCHUNK, CW, H),
        Wr_ui, bl_ui.reshape(1, H),
        s_u, c_u, h_user, Wl_iu.reshape(NCHUNK, CW, H),
        Wr_iu, bl_iu.reshape(1, H))
    sc_u, sh_u = _bn_coeffs(st_u, g_u, be_u)
    sc_i, sh_i = _bn_coeffs(st_i, g_i, be_i)
    h_user, h_item = _bn(out_u, sc_u, sh_u, out_i, sc_i, sh_i)

  return (h_user, h_item)


# speedup vs baseline: 1.1419x; 1.1419x over previous
import functools

import jax
import jax.numpy as jnp
from jax import lax
from jax.experimental import pallas as pl
from jax.experimental.pallas import tpu as pltpu
from jax.experimental.pallas import tpu_sc as plsc

N = 50000
E = 625000
D = 128
H = 128
EPS = 1e-5

NCHUNK = 4
CW = 32
NTILE = 16
BLK = 1280
NBLK = 31
E_PAD = NTILE * NBLK * BLK
PAD_ROWS = 304
ACC_N = N + PAD_ROWS
ZROWS = ACC_N // NTILE
OROWS = 3128
CNTW = 8
KB = BLK // 128


def _seg_body(h4, src2, dst2, zeros32, s_out, gi2, dsw2, rows, acc_sh, sems):
  c = lax.axis_index("c")
  s = lax.axis_index("s")
  G = NBLK * KB

  for l in range(2):
    cc = 2 * c + l
    pltpu.sync_copy(zeros32, acc_sh.at[pl.ds(ZROWS * s, ZROWS)])
    plsc.subcore_barrier()

    def fire_idx(j, p):
      row0 = (s * NBLK + j) * KB
      pltpu.async_copy(src2.at[pl.ds(row0, KB)],
                       gi2.at[pl.ds(p * KB, KB)], sems.at[p])
      pltpu.async_copy(dst2.at[pl.ds(row0, KB)],
                       dsw2.at[pl.ds(p * KB, KB)], sems.at[p])

    def wait_idx(j, p):
      row0 = (s * NBLK + j) * KB
      pltpu.make_async_copy(src2.at[pl.ds(row0, KB)],
                            gi2.at[pl.ds(p * KB, KB)], sems.at[p]).wait()
      pltpu.make_async_copy(dst2.at[pl.ds(row0, KB)],
                            dsw2.at[pl.ds(p * KB, KB)], sems.at[p]).wait()

    def transform(p):
      def outer(k, _):
        def inner(i, _):
          gi2[p * KB + k, pl.ds(i * 16, 16)] = (
              gi2[p * KB + k, pl.ds(i * 16, 16)] * NCHUNK + cc)
          return None
        lax.fori_loop(0, 8, inner, None)
        return None
      lax.fori_loop(0, KB, outer, None)

    def fire_gather(t):
      p = (t // KB) % 2
      rb = t % 3
      pltpu.async_copy(h4.at[gi2.at[p * KB + t % KB]],
                       rows.at[pl.ds(rb * 128, 128)], sems.at[2 + rb])

    def wait_gather(t):
      rb = t % 3
      pltpu.make_async_copy(h4.at[pl.ds(0, 128)],
                            rows.at[pl.ds(rb * 128, 128)],
                            sems.at[2 + rb]).wait()

    def fire_scatter(t):
      p = (t // KB) % 2
      rb = t % 3
      pltpu.async_copy(rows.at[pl.ds(rb * 128, 128)],
                       acc_sh.at[dsw2.at[p * KB + t % KB]],
                       sems.at[5 + rb], add=True)

    def wait_scatter(t):
      rb = t % 3
      pltpu.make_async_copy(h4.at[pl.ds(0, 128)],
                            rows.at[pl.ds(rb * 128, 128)],
                            sems.at[5 + rb]).wait()

    fire_idx(0, 0)
    wait_idx(0, 0)
    transform(0)
    fire_idx(1, 1)
    fire_gather(0)
    fire_gather(1)

    def body(t, _):
      t2 = t + 2

      @pl.when((t2 % KB == 0) & (t2 < G))
      def _():
        wait_idx(t2 // KB, (t2 // KB) % 2)
        transform((t2 // KB) % 2)

      wait_gather(t)
      fire_scatter(t)

      @pl.when(t2 < G)
      def _():
        @pl.when(t >= 1)
        def _():
          wait_scatter(t2)

        fire_gather(t2)

      @pl.when((t % KB == 2) & (t // KB >= 1) & (t // KB + 1 < NBLK))
      def _():
        fire_idx(t // KB + 1, (t // KB + 1) % 2)
      return None

    lax.fori_loop(0, G, body, None)
    wait_scatter(G - 3)
    wait_scatter(G - 2)
    wait_scatter(G - 1)
    plsc.subcore_barrier()

    last = N - 15 * OROWS

    @pl.when(s < 15)
    def _():
      pltpu.sync_copy(acc_sh.at[pl.ds(OROWS * s, OROWS)],
                      s_out.at[cc, pl.ds(OROWS * s, OROWS)])

    @pl.when(s == 15)
    def _():
      pltpu.sync_copy(acc_sh.at[pl.ds(OROWS * 15, last)],
                      s_out.at[cc, pl.ds(OROWS * 15, last)])
    plsc.subcore_barrier()


_seg = functools.partial(
    pl.kernel,
    mesh=plsc.VectorSubcoreMesh(core_axis_name="c", subcore_axis_name="s"),
    compiler_params=pltpu.CompilerParams(use_tc_tiling_on_sc=False),
    out_type=jax.ShapeDtypeStruct((NCHUNK, N, CW), jnp.float32),
    scratch_types=[
        pltpu.VMEM((2 * KB, 128), jnp.int32),
        pltpu.VMEM((2 * KB, 128), jnp.int32),
        pltpu.VMEM((384, CW), jnp.float32),
        pltpu.VMEM_SHARED((ACC_N, CW), jnp.float32),
        pltpu.SemaphoreType.DMA((8,)),
    ],
)(_seg_body)


def _cnt_body(dst2_ui, dst2_iu, zeros8, ones8, cui_out, ciu_out,
              dstw, ones_v, cnt_sh):
  c = lax.axis_index("c")
  s = lax.axis_index("s")

  pltpu.sync_copy(ones8, ones_v)
  pltpu.sync_copy(zeros8, cnt_sh.at[pl.ds(ZROWS * s, ZROWS)])
  plsc.subcore_barrier()

  def count(dst2):
    def block(j, _):
      row0 = (s * NBLK + j) * KB
      pltpu.sync_copy(dst2.at[pl.ds(row0, KB)], dstw)
      for k in range(KB):
        pltpu.sync_copy(ones_v, cnt_sh.at[dstw.at[k]], add=True)
      return None
    lax.fori_loop(0, NBLK, block, None)

  def copy_out(c_out):
    last = N - 15 * OROWS

    @pl.when(s < 15)
    def _():
      pltpu.sync_copy(cnt_sh.at[pl.ds(OROWS * s, OROWS)],
                      c_out.at[pl.ds(OROWS * s, OROWS)])

    @pl.when(s == 15)
    def _():
      pltpu.sync_copy(cnt_sh.at[pl.ds(OROWS * 15, last)],
                      c_out.at[pl.ds(OROWS * 15, last)])

  @pl.when(c == 0)
  def _():
    count(dst2_ui)
  @pl.when(c == 1)
  def _():
    count(dst2_iu)
  plsc.subcore_barrier()

  @pl.when(c == 0)
  def _():
    copy_out(cui_out)
  @pl.when(c == 1)
  def _():
    copy_out(ciu_out)


_cnt = functools.partial(
    pl.kernel,
    mesh=plsc.VectorSubcoreMesh(core_axis_name="c", subcore_axis_name="s"),
    compiler_params=pltpu.CompilerParams(use_tc_tiling_on_sc=False),
    out_type=[
        jax.ShapeDtypeStruct((N, CNTW), jnp.float32),
        jax.ShapeDtypeStruct((N, CNTW), jnp.float32),
    ],
    scratch_types=[
        pltpu.VMEM((KB, 128), jnp.int32),
        pltpu.VMEM((128, CNTW), jnp.float32),
        pltpu.VMEM_SHARED((ACC_N, CNTW), jnp.float32),
    ],
)(_cnt_body)


def _mm(a, b):
  return jnp.dot(a, b, preferred_element_type=jnp.float32,
                 precision=lax.Precision.HIGHEST)


def _proj_body(xu_ref, xi_ref, wu_ref, bu_ref, wi_ref, bi_ref,
               ou_ref, oi_ref):
  ou_ref[...] = _mm(xu_ref[...], wu_ref[...]) + bu_ref[...]
  oi_ref[...] = _mm(xi_ref[...], wi_ref[...]) + bi_ref[...]


_proj = pl.pallas_call(
    _proj_body,
    grid=(50,),
    in_specs=[
        pl.BlockSpec((1000, D), lambda i: (i, 0)),
        pl.BlockSpec((1000, D), lambda i: (i, 0)),
        pl.BlockSpec((D, H), lambda i: (0, 0)),
        pl.BlockSpec((1, H), lambda i: (0, 0)),
        pl.BlockSpec((D, H), lambda i: (0, 0)),
        pl.BlockSpec((1, H), lambda i: (0, 0)),
    ],
    out_specs=[
        pl.BlockSpec((1000, H), lambda i: (i, 0)),
        pl.BlockSpec((1000, H), lambda i: (i, 0)),
    ],
    out_shape=[
        jax.ShapeDtypeStruct((N, H), jnp.float32),
        jax.ShapeDtypeStruct((N, H), jnp.float32),
    ],
)


def _comb_one(i, s_ref, c_ref, h_ref, wl_ref, wr_ref, bl_ref, o_ref, st_ref):
  sv = s_ref[...]
  t = _mm(sv[0], wl_ref[0])
  for k in range(1, NCHUNK):
    t += _mm(sv[k], wl_ref[k])
  cinv = 1.0 / jnp.maximum(c_ref[...][:, :1], 1.0)
  out = t * cinv + _mm(h_ref[...], wr_ref[...]) + bl_ref[...]
  o_ref[...] = out

  @pl.when(i == 0)
  def _():
    st_ref[...] = jnp.zeros_like(st_ref)

  st_ref[0:1, :] += jnp.sum(out, axis=0, keepdims=True)
  st_ref[1:2, :] += jnp.sum(out * out, axis=0, keepdims=True)


def _comb_body(s_ref, c_ref, h_ref, wl_ref, wr_ref, bl_ref, o_ref, st_ref):
  _comb_one(pl.program_id(0), s_ref, c_ref, h_ref, wl_ref, wr_ref, bl_ref,
            o_ref, st_ref)


_comb = pl.pallas_call(
    _comb_body,
    grid=(50,),
    in_specs=[
        pl.BlockSpec((NCHUNK, 1000, CW), lambda i: (0, i, 0)),
        pl.BlockSpec((1000, CNTW), lambda i: (i, 0)),
        pl.BlockSpec((1000, D), lambda i: (i, 0)),
        pl.BlockSpec((NCHUNK, CW, H), lambda i: (0, 0, 0)),
        pl.BlockSpec((D, H), lambda i: (0, 0)),
        pl.BlockSpec((1, H), lambda i: (0, 0)),
    ],
    out_specs=[
        pl.BlockSpec((1000, H), lambda i: (i, 0)),
        pl.BlockSpec((8, H), lambda i: (0, 0)),
    ],
    out_shape=[
        jax.ShapeDtypeStruct((N, H), jnp.float32),
        jax.ShapeDtypeStruct((8, H), jnp.float32),
    ],
)


def _bn_body(xu_ref, scu_ref, shu_ref, xi_ref, sci_ref, shi_ref,
             ou_ref, oi_ref):
  ou_ref[...] = jnp.maximum(xu_ref[...] * scu_ref[...] + shu_ref[...], 0.0)
  oi_ref[...] = jnp.maximum(xi_ref[...] * sci_ref[...] + shi_ref[...], 0.0)


_bn = pl.pallas_call(
    _bn_body,
    grid=(50,),
    in_specs=[
        pl.BlockSpec((1000, H), lambda i: (i, 0)),
        pl.BlockSpec((1, H), lambda i: (0, 0)),
        pl.BlockSpec((1, H), lambda i: (0, 0)),
        pl.BlockSpec((1000, H), lambda i: (i, 0)),
        pl.BlockSpec((1, H), lambda i: (0, 0)),
        pl.BlockSpec((1, H), lambda i: (0, 0)),
    ],
    out_specs=[
        pl.BlockSpec((1000, H), lambda i: (i, 0)),
        pl.BlockSpec((1000, H), lambda i: (i, 0)),
    ],
    out_shape=[
        jax.ShapeDtypeStruct((N, H), jnp.float32),
        jax.ShapeDtypeStruct((N, H), jnp.float32),
    ],
)


def _pad_edges(edge):
  npad = E_PAD - E
  ar = jnp.arange(npad, dtype=jnp.int32)
  pad_src = (ar * 37) % N
  pad_dst = N + ar % PAD_ROWS
  src2 = jnp.concatenate([edge[0], pad_src]).reshape(-1, 128)
  dst2 = jnp.concatenate([edge[1], pad_dst]).reshape(-1, 128)
  return src2, dst2


def _bn_coeffs(st, g, be):
  mu = st[0] / N
  var = jnp.maximum(st[1] / N - mu * mu, 0.0)
  scale = g / jnp.sqrt(var + EPS)
  shift = be - mu * scale
  return scale.reshape(1, H), shift.reshape(1, H)


def kernel(x_user, x_item, edge_ui, edge_iu,
           W_in_user, b_in_user, W_in_item, b_in_item,
           Wl0_ui, bl0_ui, Wr0_ui, Wl0_iu, bl0_iu, Wr0_iu,
           g0_user, be0_user, g0_item, be0_item,
           Wl1_ui, bl1_ui, Wr1_ui, Wl1_iu, bl1_iu, Wr1_iu,
           g1_user, be1_user, g1_item, be1_item):
  src_ui, dst_ui = _pad_edges(edge_ui)
  src_iu, dst_iu = _pad_edges(edge_iu)
  zeros32 = jnp.zeros((ZROWS, CW), jnp.float32)
  zeros8 = jnp.zeros((ZROWS, CNTW), jnp.float32)
  ones8 = jnp.ones((128, CNTW), jnp.float32)

  c_i, c_u = _cnt(dst_ui, dst_iu, zeros8, ones8)

  h_user, h_item = _proj(x_user, x_item,
                         W_in_user, b_in_user.reshape(1, H),
                         W_in_item, b_in_item.reshape(1, H))

  params = [
      (Wl0_ui, bl0_ui, Wr0_ui, Wl0_iu, bl0_iu, Wr0_iu,
       g0_user, be0_user, g0_item, be0_item),
      (Wl1_ui, bl1_ui, Wr1_ui, Wl1_iu, bl1_iu, Wr1_iu,
       g1_user, be1_user, g1_item, be1_item),
  ]
  for (Wl_ui, bl_ui, Wr_ui, Wl_iu, bl_iu, Wr_iu,
       g_u, be_u, g_i, be_i) in params:
    s_i = _seg(h_user.reshape(N * NCHUNK, CW), src_ui, dst_ui, zeros32)
    s_u = _seg(h_item.reshape(N * NCHUNK, CW), src_iu, dst_iu, zeros32)
    out_i, st_i = _comb(s_i, c_i, h_item, Wl_ui.reshape(NCHUNK, CW, H),
                        Wr_ui, bl_ui.reshape(1, H))
    out_u, st_u = _comb(s_u, c_u, h_user, Wl_iu.reshape(NCHUNK, CW, H),
                        Wr_iu, bl_iu.reshape(1, H))
    sc_u, sh_u = _bn_coeffs(st_u, g_u, be_u)
    sc_i, sh_i = _bn_coeffs(st_i, g_i, be_i)
    h_user, h_item = _bn(out_u, sc_u, sh_u, out_i, sc_i, sh_i)

  return (h_user, h_item)


# speedup vs baseline: 1.4385x; 1.2597x over previous
import functools

import jax
import jax.numpy as jnp
from jax import lax
from jax.experimental import pallas as pl
from jax.experimental.pallas import tpu as pltpu
from jax.experimental.pallas import tpu_sc as plsc

N = 50000
E = 625000
D = 128
H = 128
EPS = 1e-5

NCHUNK = 4
CW = 32
NTILE = 16
BLK = 1280
NBLK = 31
E_PAD = NTILE * NBLK * BLK
PAD_ROWS = 304
ACC_N = N + PAD_ROWS
ZROWS = ACC_N // NTILE
OROWS = 3128
CNTW = 8
KB = BLK // 128


def _seg_body(h4, src2, dst2, zeros32, s_out, gi2, dsw2, rows, acc_sh, sems):
  c = lax.axis_index("c")
  s = lax.axis_index("s")
  G = NBLK * KB

  for l in range(2):
    cc = 2 * c + l
    pltpu.sync_copy(zeros32, acc_sh.at[pl.ds(ZROWS * s, ZROWS)])
    plsc.subcore_barrier()

    def fire_idx(j, p):
      row0 = (s * NBLK + j) * KB
      pltpu.async_copy(src2.at[pl.ds(row0, KB)],
                       gi2.at[pl.ds(p * KB, KB)], sems.at[p])
      pltpu.async_copy(dst2.at[pl.ds(row0, KB)],
                       dsw2.at[pl.ds(p * KB, KB)], sems.at[p])

    def wait_idx(j, p):
      row0 = (s * NBLK + j) * KB
      pltpu.make_async_copy(src2.at[pl.ds(row0, KB)],
                            gi2.at[pl.ds(p * KB, KB)], sems.at[p]).wait()
      pltpu.make_async_copy(dst2.at[pl.ds(row0, KB)],
                            dsw2.at[pl.ds(p * KB, KB)], sems.at[p]).wait()

    def transform(p):
      def outer(k, _):
        def inner(i, _):
          gi2[p * KB + k, pl.ds(i * 16, 16)] = (
              gi2[p * KB + k, pl.ds(i * 16, 16)] * NCHUNK + cc)
          return None
        lax.fori_loop(0, 8, inner, None)
        return None
      lax.fori_loop(0, KB, outer, None)

    def fire_gather(t):
      p = (t // KB) % 2
      rb = t % 3
      pltpu.async_copy(h4.at[gi2.at[p * KB + t % KB]],
                       rows.at[pl.ds(rb * 128, 128)], sems.at[2 + rb])

    def wait_gather(t):
      rb = t % 3
      pltpu.make_async_copy(h4.at[pl.ds(0, 128)],
                            rows.at[pl.ds(rb * 128, 128)],
                            sems.at[2 + rb]).wait()

    def fire_scatter(t):
      p = (t // KB) % 2
      rb = t % 3
      pltpu.async_copy(rows.at[pl.ds(rb * 128, 128)],
                       acc_sh.at[dsw2.at[p * KB + t % KB]],
                       sems.at[5 + rb], add=True)

    def wait_scatter(t):
      rb = t % 3
      pltpu.make_async_copy(h4.at[pl.ds(0, 128)],
                            rows.at[pl.ds(rb * 128, 128)],
                            sems.at[5 + rb]).wait()

    fire_idx(0, 0)
    wait_idx(0, 0)
    transform(0)
    fire_idx(1, 1)
    fire_gather(0)
    fire_gather(1)

    def body(t, _):
      t2 = t + 2

      @pl.when((t2 % KB == 0) & (t2 < G))
      def _():
        wait_idx(t2 // KB, (t2 // KB) % 2)
        transform((t2 // KB) % 2)

      wait_gather(t)
      fire_scatter(t)

      @pl.when(t2 < G)
      def _():
        @pl.when(t >= 1)
        def _():
          wait_scatter(t2)

        fire_gather(t2)

      @pl.when((t % KB == 2) & (t // KB >= 1) & (t // KB + 1 < NBLK))
      def _():
        fire_idx(t // KB + 1, (t // KB + 1) % 2)
      return None

    lax.fori_loop(0, G, body, None)
    wait_scatter(G - 3)
    wait_scatter(G - 2)
    wait_scatter(G - 1)
    plsc.subcore_barrier()

    last = N - 15 * OROWS

    @pl.when(s < 15)
    def _():
      pltpu.sync_copy(acc_sh.at[pl.ds(OROWS * s, OROWS)],
                      s_out.at[cc, pl.ds(OROWS * s, OROWS)])

    @pl.when(s == 15)
    def _():
      pltpu.sync_copy(acc_sh.at[pl.ds(OROWS * 15, last)],
                      s_out.at[cc, pl.ds(OROWS * 15, last)])
    plsc.subcore_barrier()


_seg = functools.partial(
    pl.kernel,
    mesh=plsc.VectorSubcoreMesh(core_axis_name="c", subcore_axis_name="s"),
    compiler_params=pltpu.CompilerParams(use_tc_tiling_on_sc=False),
    out_type=jax.ShapeDtypeStruct((NCHUNK, N, CW), jnp.float32),
    scratch_types=[
        pltpu.VMEM((2 * KB, 128), jnp.int32),
        pltpu.VMEM((2 * KB, 128), jnp.int32),
        pltpu.VMEM((384, CW), jnp.float32),
        pltpu.VMEM_SHARED((ACC_N, CW), jnp.float32),
        pltpu.SemaphoreType.DMA((8,)),
    ],
)(_seg_body)


def _cnt_body(dst2_ui, dst2_iu, zeros8, ones8, cui_out, ciu_out,
              dstw, ones_v, cnt_sh):
  c = lax.axis_index("c")
  s = lax.axis_index("s")

  pltpu.sync_copy(ones8, ones_v)
  pltpu.sync_copy(zeros8, cnt_sh.at[pl.ds(ZROWS * s, ZROWS)])
  plsc.subcore_barrier()

  def count(dst2):
    def block(j, _):
      row0 = (s * NBLK + j) * KB
      pltpu.sync_copy(dst2.at[pl.ds(row0, KB)], dstw)
      for k in range(KB):
        pltpu.sync_copy(ones_v, cnt_sh.at[dstw.at[k]], add=True)
      return None
    lax.fori_loop(0, NBLK, block, None)

  def copy_out(c_out):
    last = N - 15 * OROWS

    @pl.when(s < 15)
    def _():
      pltpu.sync_copy(cnt_sh.at[pl.ds(OROWS * s, OROWS)],
                      c_out.at[pl.ds(OROWS * s, OROWS)])

    @pl.when(s == 15)
    def _():
      pltpu.sync_copy(cnt_sh.at[pl.ds(OROWS * 15, last)],
                      c_out.at[pl.ds(OROWS * 15, last)])

  @pl.when(c == 0)
  def _():
    count(dst2_ui)
  @pl.when(c == 1)
  def _():
    count(dst2_iu)
  plsc.subcore_barrier()

  @pl.when(c == 0)
  def _():
    copy_out(cui_out)
  @pl.when(c == 1)
  def _():
    copy_out(ciu_out)


_cnt = functools.partial(
    pl.kernel,
    mesh=plsc.VectorSubcoreMesh(core_axis_name="c", subcore_axis_name="s"),
    compiler_params=pltpu.CompilerParams(use_tc_tiling_on_sc=False),
    out_type=[
        jax.ShapeDtypeStruct((N, CNTW), jnp.float32),
        jax.ShapeDtypeStruct((N, CNTW), jnp.float32),
    ],
    scratch_types=[
        pltpu.VMEM((KB, 128), jnp.int32),
        pltpu.VMEM((128, CNTW), jnp.float32),
        pltpu.VMEM_SHARED((ACC_N, CNTW), jnp.float32),
    ],
)(_cnt_body)


def _mm(a, b):
  return jnp.dot(a, b, preferred_element_type=jnp.float32,
                 precision=lax.Precision.HIGHEST)


def _proj_body(x_ref, w_ref, b_ref, o_ref):
  o_ref[...] = _mm(x_ref[...], w_ref[...]) + b_ref[...]


_proj = pl.pallas_call(
    _proj_body,
    grid=(50,),
    in_specs=[
        pl.BlockSpec((1000, D), lambda i: (i, 0)),
        pl.BlockSpec((D, H), lambda i: (0, 0)),
        pl.BlockSpec((1, H), lambda i: (0, 0)),
    ],
    out_specs=pl.BlockSpec((1000, H), lambda i: (i, 0)),
    out_shape=jax.ShapeDtypeStruct((N, H), jnp.float32),
)


def _comb_one(i, s_ref, c_ref, h_ref, wl_ref, wr_ref, bl_ref, o_ref, st_ref):
  sv = s_ref[...]
  t = _mm(sv[0], wl_ref[0])
  for k in range(1, NCHUNK):
    t += _mm(sv[k], wl_ref[k])
  cinv = 1.0 / jnp.maximum(c_ref[...][:, :1], 1.0)
  out = t * cinv + _mm(h_ref[...], wr_ref[...]) + bl_ref[...]
  o_ref[...] = out

  @pl.when(i == 0)
  def _():
    st_ref[...] = jnp.zeros_like(st_ref)

  st_ref[0:1, :] += jnp.sum(out, axis=0, keepdims=True)
  st_ref[1:2, :] += jnp.sum(out * out, axis=0, keepdims=True)


def _comb_body(s_ref, c_ref, h_ref, wl_ref, wr_ref, bl_ref, o_ref, st_ref):
  _comb_one(pl.program_id(0), s_ref, c_ref, h_ref, wl_ref, wr_ref, bl_ref,
            o_ref, st_ref)


_comb = pl.pallas_call(
    _comb_body,
    grid=(50,),
    in_specs=[
        pl.BlockSpec((NCHUNK, 1000, CW), lambda i: (0, i, 0)),
        pl.BlockSpec((1000, CNTW), lambda i: (i, 0)),
        pl.BlockSpec((1000, D), lambda i: (i, 0)),
        pl.BlockSpec((NCHUNK, CW, H), lambda i: (0, 0, 0)),
        pl.BlockSpec((D, H), lambda i: (0, 0)),
        pl.BlockSpec((1, H), lambda i: (0, 0)),
    ],
    out_specs=[
        pl.BlockSpec((1000, H), lambda i: (i, 0)),
        pl.BlockSpec((8, H), lambda i: (0, 0)),
    ],
    out_shape=[
        jax.ShapeDtypeStruct((N, H), jnp.float32),
        jax.ShapeDtypeStruct((8, H), jnp.float32),
    ],
)


def _bn_body(x_ref, sc_ref, sh_ref, o_ref):
  o_ref[...] = jnp.maximum(x_ref[...] * sc_ref[...] + sh_ref[...], 0.0)


_bn = pl.pallas_call(
    _bn_body,
    grid=(50,),
    in_specs=[
        pl.BlockSpec((1000, H), lambda i: (i, 0)),
        pl.BlockSpec((1, H), lambda i: (0, 0)),
        pl.BlockSpec((1, H), lambda i: (0, 0)),
    ],
    out_specs=pl.BlockSpec((1000, H), lambda i: (i, 0)),
    out_shape=jax.ShapeDtypeStruct((N, H), jnp.float32),
)


def _pad_edges(edge):
  npad = E_PAD - E
  ar = jnp.arange(npad, dtype=jnp.int32)
  pad_src = (ar * 37) % N
  pad_dst = N + ar % PAD_ROWS
  src2 = jnp.concatenate([edge[0], pad_src]).reshape(-1, 128)
  dst2 = jnp.concatenate([edge[1], pad_dst]).reshape(-1, 128)
  return src2, dst2


def _bn_coeffs(st, g, be):
  mu = st[0] / N
  var = jnp.maximum(st[1] / N - mu * mu, 0.0)
  scale = g / jnp.sqrt(var + EPS)
  shift = be - mu * scale
  return scale.reshape(1, H), shift.reshape(1, H)


def kernel(x_user, x_item, edge_ui, edge_iu,
           W_in_user, b_in_user, W_in_item, b_in_item,
           Wl0_ui, bl0_ui, Wr0_ui, Wl0_iu, bl0_iu, Wr0_iu,
           g0_user, be0_user, g0_item, be0_item,
           Wl1_ui, bl1_ui, Wr1_ui, Wl1_iu, bl1_iu, Wr1_iu,
           g1_user, be1_user, g1_item, be1_item):
  src_ui, dst_ui = _pad_edges(edge_ui)
  src_iu, dst_iu = _pad_edges(edge_iu)
  zeros32 = jnp.zeros((ZROWS, CW), jnp.float32)
  zeros8 = jnp.zeros((ZROWS, CNTW), jnp.float32)
  ones8 = jnp.ones((128, CNTW), jnp.float32)

  c_i, c_u = _cnt(dst_ui, dst_iu, zeros8, ones8)

  h_user = _proj(x_user, W_in_user, b_in_user.reshape(1, H))
  h_item = _proj(x_item, W_in_item, b_in_item.reshape(1, H))

  params = [
      (Wl0_ui, bl0_ui, Wr0_ui, Wl0_iu, bl0_iu, Wr0_iu,
       g0_user, be0_user, g0_item, be0_item),
      (Wl1_ui, bl1_ui, Wr1_ui, Wl1_iu, bl1_iu, Wr1_iu,
       g1_user, be1_user, g1_item, be1_item),
  ]
  for (Wl_ui, bl_ui, Wr_ui, Wl_iu, bl_iu, Wr_iu,
       g_u, be_u, g_i, be_i) in params:
    s_i = _seg(h_user.reshape(N * NCHUNK, CW), src_ui, dst_ui, zeros32)
    s_u = _seg(h_item.reshape(N * NCHUNK, CW), src_iu, dst_iu, zeros32)
    out_i, st_i = _comb(s_i, c_i, h_item, Wl_ui.reshape(NCHUNK, CW, H),
                        Wr_ui, bl_ui.reshape(1, H))
    out_u, st_u = _comb(s_u, c_u, h_user, Wl_iu.reshape(NCHUNK, CW, H),
                        Wr_iu, bl_iu.reshape(1, H))
    sc_u, sh_u = _bn_coeffs(st_u, g_u, be_u)
    sc_i, sh_i = _bn_coeffs(st_i, g_i, be_i)
    h_user = _bn(out_u, sc_u, sh_u)
    h_item = _bn(out_i, sc_i, sh_i)

  return (h_user, h_item)


# speedup vs baseline: 1.5379x; 1.0691x over previous
import functools

import jax
import jax.numpy as jnp
from jax import lax
from jax.experimental import pallas as pl
from jax.experimental.pallas import tpu as pltpu
from jax.experimental.pallas import tpu_sc as plsc

N = 50000
E = 625000
D = 128
H = 128
EPS = 1e-5

NCHUNK = 4
CW = 32
NTILE = 16
BLK = 1280
NBLK = 31
E_PAD = NTILE * NBLK * BLK
PAD_ROWS = 304
ACC_N = N + PAD_ROWS
ZROWS = ACC_N // NTILE
OROWS = 3128
CNTW = 8
KB = BLK // 128


def _seg_body(h4, src2, dst2, zeros32, s_out, gi2, dsw2, rows, acc_sh, sems):
  c = lax.axis_index("c")
  s = lax.axis_index("s")
  G = NBLK * KB

  for l in range(2):
    cc = 2 * c + l
    pltpu.sync_copy(zeros32, acc_sh.at[pl.ds(ZROWS * s, ZROWS)])
    plsc.subcore_barrier()

    def fire_idx(j, p):
      row0 = (s * NBLK + j) * KB
      pltpu.async_copy(src2.at[pl.ds(row0, KB)],
                       gi2.at[pl.ds(p * KB, KB)], sems.at[p])
      pltpu.async_copy(dst2.at[pl.ds(row0, KB)],
                       dsw2.at[pl.ds(p * KB, KB)], sems.at[p])

    def wait_idx(j, p):
      row0 = (s * NBLK + j) * KB
      pltpu.make_async_copy(src2.at[pl.ds(row0, KB)],
                            gi2.at[pl.ds(p * KB, KB)], sems.at[p]).wait()
      pltpu.make_async_copy(dst2.at[pl.ds(row0, KB)],
                            dsw2.at[pl.ds(p * KB, KB)], sems.at[p]).wait()

    def transform(p):
      def outer(k, _):
        def inner(i, _):
          gi2[p * KB + k, pl.ds(i * 16, 16)] = (
              gi2[p * KB + k, pl.ds(i * 16, 16)] * NCHUNK + cc)
          return None
        lax.fori_loop(0, 8, inner, None)
        return None
      lax.fori_loop(0, KB, outer, None)

    def fire_gather(t):
      p = (t // KB) % 2
      rb = t % 4
      pltpu.async_copy(h4.at[gi2.at[p * KB + t % KB]],
                       rows.at[pl.ds(rb * 128, 128)], sems.at[2 + rb])

    def wait_gather(t):
      rb = t % 4
      pltpu.make_async_copy(h4.at[pl.ds(0, 128)],
                            rows.at[pl.ds(rb * 128, 128)],
                            sems.at[2 + rb]).wait()

    def fire_scatter(t):
      p = (t // KB) % 2
      rb = t % 4
      pltpu.async_copy(rows.at[pl.ds(rb * 128, 128)],
                       acc_sh.at[dsw2.at[p * KB + t % KB]],
                       sems.at[6 + rb], add=True)

    def wait_scatter(t):
      rb = t % 4
      pltpu.make_async_copy(h4.at[pl.ds(0, 128)],
                            rows.at[pl.ds(rb * 128, 128)],
                            sems.at[6 + rb]).wait()

    fire_idx(0, 0)
    wait_idx(0, 0)
    transform(0)
    fire_idx(1, 1)
    fire_gather(0)
    fire_gather(1)
    fire_gather(2)

    def body(t, _):
      t3 = t + 3

      @pl.when((t3 % KB == 0) & (t3 < G))
      def _():
        wait_idx(t3 // KB, (t3 // KB) % 2)
        transform((t3 // KB) % 2)

      wait_gather(t)
      fire_scatter(t)

      @pl.when(t3 < G)
      def _():
        @pl.when(t >= 1)
        def _():
          wait_scatter(t3)

        fire_gather(t3)

      @pl.when((t % KB == 2) & (t // KB >= 1) & (t // KB + 1 < NBLK))
      def _():
        fire_idx(t // KB + 1, (t // KB + 1) % 2)
      return None

    lax.fori_loop(0, G, body, None)
    wait_scatter(G - 4)
    wait_scatter(G - 3)
    wait_scatter(G - 2)
    wait_scatter(G - 1)
    plsc.subcore_barrier()

    last = N - 15 * OROWS

    @pl.when(s < 15)
    def _():
      pltpu.sync_copy(acc_sh.at[pl.ds(OROWS * s, OROWS)],
                      s_out.at[cc, pl.ds(OROWS * s, OROWS)])

    @pl.when(s == 15)
    def _():
      pltpu.sync_copy(acc_sh.at[pl.ds(OROWS * 15, last)],
                      s_out.at[cc, pl.ds(OROWS * 15, last)])
    plsc.subcore_barrier()


_seg = functools.partial(
    pl.kernel,
    mesh=plsc.VectorSubcoreMesh(core_axis_name="c", subcore_axis_name="s"),
    compiler_params=pltpu.CompilerParams(use_tc_tiling_on_sc=False),
    out_type=jax.ShapeDtypeStruct((NCHUNK, N, CW), jnp.float32),
    scratch_types=[
        pltpu.VMEM((2 * KB, 128), jnp.int32),
        pltpu.VMEM((2 * KB, 128), jnp.int32),
        pltpu.VMEM((512, CW), jnp.float32),
        pltpu.VMEM_SHARED((ACC_N, CW), jnp.float32),
        pltpu.SemaphoreType.DMA((10,)),
    ],
)(_seg_body)


def _cnt_body(dst2_ui, dst2_iu, zeros8, ones8, cui_out, ciu_out,
              dstw, ones_v, cnt_sh):
  c = lax.axis_index("c")
  s = lax.axis_index("s")

  pltpu.sync_copy(ones8, ones_v)
  pltpu.sync_copy(zeros8, cnt_sh.at[pl.ds(ZROWS * s, ZROWS)])
  plsc.subcore_barrier()

  def count(dst2):
    def block(j, _):
      row0 = (s * NBLK + j) * KB
      pltpu.sync_copy(dst2.at[pl.ds(row0, KB)], dstw)
      for k in range(KB):
        pltpu.sync_copy(ones_v, cnt_sh.at[dstw.at[k]], add=True)
      return None
    lax.fori_loop(0, NBLK, block, None)

  def copy_out(c_out):
    last = N - 15 * OROWS

    @pl.when(s < 15)
    def _():
      pltpu.sync_copy(cnt_sh.at[pl.ds(OROWS * s, OROWS)],
                      c_out.at[pl.ds(OROWS * s, OROWS)])

    @pl.when(s == 15)
    def _():
      pltpu.sync_copy(cnt_sh.at[pl.ds(OROWS * 15, last)],
                      c_out.at[pl.ds(OROWS * 15, last)])

  @pl.when(c == 0)
  def _():
    count(dst2_ui)
  @pl.when(c == 1)
  def _():
    count(dst2_iu)
  plsc.subcore_barrier()

  @pl.when(c == 0)
  def _():
    copy_out(cui_out)
  @pl.when(c == 1)
  def _():
    copy_out(ciu_out)


_cnt = functools.partial(
    pl.kernel,
    mesh=plsc.VectorSubcoreMesh(core_axis_name="c", subcore_axis_name="s"),
    compiler_params=pltpu.CompilerParams(use_tc_tiling_on_sc=False),
    out_type=[
        jax.ShapeDtypeStruct((N, CNTW), jnp.float32),
        jax.ShapeDtypeStruct((N, CNTW), jnp.float32),
    ],
    scratch_types=[
        pltpu.VMEM((KB, 128), jnp.int32),
        pltpu.VMEM((128, CNTW), jnp.float32),
        pltpu.VMEM_SHARED((ACC_N, CNTW), jnp.float32),
    ],
)(_cnt_body)


def _mm(a, b):
  return jnp.dot(a, b, preferred_element_type=jnp.float32,
                 precision=lax.Precision.HIGHEST)


def _proj_body(x_ref, w_ref, b_ref, o_ref):
  o_ref[...] = _mm(x_ref[...], w_ref[...]) + b_ref[...]


_proj = pl.pallas_call(
    _proj_body,
    grid=(50,),
    in_specs=[
        pl.BlockSpec((1000, D), lambda i: (i, 0)),
        pl.BlockSpec((D, H), lambda i: (0, 0)),
        pl.BlockSpec((1, H), lambda i: (0, 0)),
    ],
    out_specs=pl.BlockSpec((1000, H), lambda i: (i, 0)),
    out_shape=jax.ShapeDtypeStruct((N, H), jnp.float32),
)


def _comb_one(i, s_ref, c_ref, h_ref, wl_ref, wr_ref, bl_ref, o_ref, st_ref):
  sv = s_ref[...]
  t = _mm(sv[0], wl_ref[0])
  for k in range(1, NCHUNK):
    t += _mm(sv[k], wl_ref[k])
  cinv = 1.0 / jnp.maximum(c_ref[...][:, :1], 1.0)
  out = t * cinv + _mm(h_ref[...], wr_ref[...]) + bl_ref[...]
  o_ref[...] = out

  @pl.when(i == 0)
  def _():
    st_ref[...] = jnp.zeros_like(st_ref)

  st_ref[0:1, :] += jnp.sum(out, axis=0, keepdims=True)
  st_ref[1:2, :] += jnp.sum(out * out, axis=0, keepdims=True)


def _comb_body(s_ref, c_ref, h_ref, wl_ref, wr_ref, bl_ref, o_ref, st_ref):
  _comb_one(pl.program_id(0), s_ref, c_ref, h_ref, wl_ref, wr_ref, bl_ref,
            o_ref, st_ref)


_comb = pl.pallas_call(
    _comb_body,
    grid=(50,),
    in_specs=[
        pl.BlockSpec((NCHUNK, 1000, CW), lambda i: (0, i, 0)),
        pl.BlockSpec((1000, CNTW), lambda i: (i, 0)),
        pl.BlockSpec((1000, D), lambda i: (i, 0)),
        pl.BlockSpec((NCHUNK, CW, H), lambda i: (0, 0, 0)),
        pl.BlockSpec((D, H), lambda i: (0, 0)),
        pl.BlockSpec((1, H), lambda i: (0, 0)),
    ],
    out_specs=[
        pl.BlockSpec((1000, H), lambda i: (i, 0)),
        pl.BlockSpec((8, H), lambda i: (0, 0)),
    ],
    out_shape=[
        jax.ShapeDtypeStruct((N, H), jnp.float32),
        jax.ShapeDtypeStruct((8, H), jnp.float32),
    ],
)


def _bn_body(x_ref, sc_ref, sh_ref, o_ref):
  o_ref[...] = jnp.maximum(x_ref[...] * sc_ref[...] + sh_ref[...], 0.0)


_bn = pl.pallas_call(
    _bn_body,
    grid=(50,),
    in_specs=[
        pl.BlockSpec((1000, H), lambda i: (i, 0)),
        pl.BlockSpec((1, H), lambda i: (0, 0)),
        pl.BlockSpec((1, H), lambda i: (0, 0)),
    ],
    out_specs=pl.BlockSpec((1000, H), lambda i: (i, 0)),
    out_shape=jax.ShapeDtypeStruct((N, H), jnp.float32),
)


def _pad_edges(edge):
  npad = E_PAD - E
  ar = jnp.arange(npad, dtype=jnp.int32)
  pad_src = (ar * 37) % N
  pad_dst = N + ar % PAD_ROWS
  src2 = jnp.concatenate([edge[0], pad_src]).reshape(-1, 128)
  dst2 = jnp.concatenate([edge[1], pad_dst]).reshape(-1, 128)
  return src2, dst2


def _bn_coeffs(st, g, be):
  mu = st[0] / N
  var = jnp.maximum(st[1] / N - mu * mu, 0.0)
  scale = g / jnp.sqrt(var + EPS)
  shift = be - mu * scale
  return scale.reshape(1, H), shift.reshape(1, H)


def kernel(x_user, x_item, edge_ui, edge_iu,
           W_in_user, b_in_user, W_in_item, b_in_item,
           Wl0_ui, bl0_ui, Wr0_ui, Wl0_iu, bl0_iu, Wr0_iu,
           g0_user, be0_user, g0_item, be0_item,
           Wl1_ui, bl1_ui, Wr1_ui, Wl1_iu, bl1_iu, Wr1_iu,
           g1_user, be1_user, g1_item, be1_item):
  src_ui, dst_ui = _pad_edges(edge_ui)
  src_iu, dst_iu = _pad_edges(edge_iu)
  zeros32 = jnp.zeros((ZROWS, CW), jnp.float32)
  zeros8 = jnp.zeros((ZROWS, CNTW), jnp.float32)
  ones8 = jnp.ones((128, CNTW), jnp.float32)

  c_i, c_u = _cnt(dst_ui, dst_iu, zeros8, ones8)

  h_user = _proj(x_user, W_in_user, b_in_user.reshape(1, H))
  h_item = _proj(x_item, W_in_item, b_in_item.reshape(1, H))

  params = [
      (Wl0_ui, bl0_ui, Wr0_ui, Wl0_iu, bl0_iu, Wr0_iu,
       g0_user, be0_user, g0_item, be0_item),
      (Wl1_ui, bl1_ui, Wr1_ui, Wl1_iu, bl1_iu, Wr1_iu,
       g1_user, be1_user, g1_item, be1_item),
  ]
  for (Wl_ui, bl_ui, Wr_ui, Wl_iu, bl_iu, Wr_iu,
       g_u, be_u, g_i, be_i) in params:
    s_i = _seg(h_user.reshape(N * NCHUNK, CW), src_ui, dst_ui, zeros32)
    s_u = _seg(h_item.reshape(N * NCHUNK, CW), src_iu, dst_iu, zeros32)
    out_i, st_i = _comb(s_i, c_i, h_item, Wl_ui.reshape(NCHUNK, CW, H),
                        Wr_ui, bl_ui.reshape(1, H))
    out_u, st_u = _comb(s_u, c_u, h_user, Wl_iu.reshape(NCHUNK, CW, H),
                        Wr_iu, bl_iu.reshape(1, H))
    sc_u, sh_u = _bn_coeffs(st_u, g_u, be_u)
    sc_i, sh_i = _bn_coeffs(st_i, g_i, be_i)
    h_user = _bn(out_u, sc_u, sh_u)
    h_item = _bn(out_i, sc_i, sh_i)

  return (h_user, h_item)


# speedup vs baseline: 1.5499x; 1.0078x over previous
import functools

import jax
import jax.numpy as jnp
from jax import lax
from jax.experimental import pallas as pl
from jax.experimental.pallas import tpu as pltpu
from jax.experimental.pallas import tpu_sc as plsc

N = 50000
E = 625000
D = 128
H = 128
EPS = 1e-5

NCHUNK = 4
CW = 32
NTILE = 16
BLK = 1280
NBLK = 31
E_PAD = NTILE * NBLK * BLK
PAD_ROWS = 304
ACC_N = N + PAD_ROWS
ZROWS = ACC_N // NTILE
OROWS = 3128
CNTW = 8
KB = BLK // 128


def _seg_body(h4, src2, dst2, zeros32, s_out, gi2, dsw2, rows, acc_sh, sems):
  c = lax.axis_index("c")
  s = lax.axis_index("s")
  G = NBLK * KB

  for l in range(2):
    cc = 2 * c + l
    pltpu.sync_copy(zeros32, acc_sh.at[pl.ds(ZROWS * s, ZROWS)])
    plsc.subcore_barrier()

    def fire_idx(j, p):
      row0 = (s * NBLK + j) * KB
      pltpu.async_copy(src2.at[pl.ds(row0, KB)],
                       gi2.at[pl.ds(p * KB, KB)], sems.at[p])
      pltpu.async_copy(dst2.at[pl.ds(row0, KB)],
                       dsw2.at[pl.ds(p * KB, KB)], sems.at[p])

    def wait_idx(j, p):
      row0 = (s * NBLK + j) * KB
      pltpu.make_async_copy(src2.at[pl.ds(row0, KB)],
                            gi2.at[pl.ds(p * KB, KB)], sems.at[p]).wait()
      pltpu.make_async_copy(dst2.at[pl.ds(row0, KB)],
                            dsw2.at[pl.ds(p * KB, KB)], sems.at[p]).wait()

    def transform(p):
      def outer(k, _):
        def inner(i, _):
          gi2[p * KB + k, pl.ds(i * 16, 16)] = (
              gi2[p * KB + k, pl.ds(i * 16, 16)] * NCHUNK + cc)
          return None
        lax.fori_loop(0, 8, inner, None)
        return None
      lax.fori_loop(0, KB, outer, None)

    def fire_gather(t):
      p = (t // KB) % 2
      rb = t % 6
      pltpu.async_copy(h4.at[gi2.at[p * KB + t % KB]],
                       rows.at[pl.ds(rb * 128, 128)], sems.at[2 + rb])

    def wait_gather(t):
      rb = t % 6
      pltpu.make_async_copy(h4.at[pl.ds(0, 128)],
                            rows.at[pl.ds(rb * 128, 128)],
                            sems.at[2 + rb]).wait()

    def fire_scatter(t):
      p = (t // KB) % 2
      rb = t % 6
      pltpu.async_copy(rows.at[pl.ds(rb * 128, 128)],
                       acc_sh.at[dsw2.at[p * KB + t % KB]],
                       sems.at[8 + rb], add=True)

    def wait_scatter(t):
      rb = t % 6
      pltpu.make_async_copy(h4.at[pl.ds(0, 128)],
                            rows.at[pl.ds(rb * 128, 128)],
                            sems.at[8 + rb]).wait()

    fire_idx(0, 0)
    wait_idx(0, 0)
    transform(0)
    fire_idx(1, 1)
    fire_gather(0)
    fire_gather(1)
    fire_gather(2)
    fire_gather(3)
    fire_gather(4)

    def body(t, _):
      t3 = t + 5

      @pl.when((t3 % KB == 0) & (t3 < G))
      def _():
        wait_idx(t3 // KB, (t3 // KB) % 2)
        transform((t3 // KB) % 2)

      wait_gather(t)
      fire_scatter(t)

      @pl.when(t3 < G)
      def _():
        @pl.when(t >= 1)
        def _():
          wait_scatter(t3)

        fire_gather(t3)

      @pl.when((t % KB == 2) & (t // KB >= 1) & (t // KB + 1 < NBLK))
      def _():
        fire_idx(t // KB + 1, (t // KB + 1) % 2)
      return None

    lax.fori_loop(0, G, body, None)
    wait_scatter(G - 6)
    wait_scatter(G - 5)
    wait_scatter(G - 4)
    wait_scatter(G - 3)
    wait_scatter(G - 2)
    wait_scatter(G - 1)
    plsc.subcore_barrier()

    last = N - 15 * OROWS

    @pl.when(s < 15)
    def _():
      pltpu.sync_copy(acc_sh.at[pl.ds(OROWS * s, OROWS)],
                      s_out.at[cc, pl.ds(OROWS * s, OROWS)])

    @pl.when(s == 15)
    def _():
      pltpu.sync_copy(acc_sh.at[pl.ds(OROWS * 15, last)],
                      s_out.at[cc, pl.ds(OROWS * 15, last)])
    plsc.subcore_barrier()


_seg = functools.partial(
    pl.kernel,
    mesh=plsc.VectorSubcoreMesh(core_axis_name="c", subcore_axis_name="s"),
    compiler_params=pltpu.CompilerParams(use_tc_tiling_on_sc=False),
    out_type=jax.ShapeDtypeStruct((NCHUNK, N, CW), jnp.float32),
    scratch_types=[
        pltpu.VMEM((2 * KB, 128), jnp.int32),
        pltpu.VMEM((2 * KB, 128), jnp.int32),
        pltpu.VMEM((768, CW), jnp.float32),
        pltpu.VMEM_SHARED((ACC_N, CW), jnp.float32),
        pltpu.SemaphoreType.DMA((14,)),
    ],
)(_seg_body)


def _cnt_body(dst2_ui, dst2_iu, zeros8, ones8, cui_out, ciu_out,
              dstw, ones_v, cnt_sh):
  c = lax.axis_index("c")
  s = lax.axis_index("s")

  pltpu.sync_copy(ones8, ones_v)
  pltpu.sync_copy(zeros8, cnt_sh.at[pl.ds(ZROWS * s, ZROWS)])
  plsc.subcore_barrier()

  def count(dst2):
    def block(j, _):
      row0 = (s * NBLK + j) * KB
      pltpu.sync_copy(dst2.at[pl.ds(row0, KB)], dstw)
      for k in range(KB):
        pltpu.sync_copy(ones_v, cnt_sh.at[dstw.at[k]], add=True)
      return None
    lax.fori_loop(0, NBLK, block, None)

  def copy_out(c_out):
    last = N - 15 * OROWS

    @pl.when(s < 15)
    def _():
      pltpu.sync_copy(cnt_sh.at[pl.ds(OROWS * s, OROWS)],
                      c_out.at[pl.ds(OROWS * s, OROWS)])

    @pl.when(s == 15)
    def _():
      pltpu.sync_copy(cnt_sh.at[pl.ds(OROWS * 15, last)],
                      c_out.at[pl.ds(OROWS * 15, last)])

  @pl.when(c == 0)
  def _():
    count(dst2_ui)
  @pl.when(c == 1)
  def _():
    count(dst2_iu)
  plsc.subcore_barrier()

  @pl.when(c == 0)
  def _():
    copy_out(cui_out)
  @pl.when(c == 1)
  def _():
    copy_out(ciu_out)


_cnt = functools.partial(
    pl.kernel,
    mesh=plsc.VectorSubcoreMesh(core_axis_name="c", subcore_axis_name="s"),
    compiler_params=pltpu.CompilerParams(use_tc_tiling_on_sc=False),
    out_type=[
        jax.ShapeDtypeStruct((N, CNTW), jnp.float32),
        jax.ShapeDtypeStruct((N, CNTW), jnp.float32),
    ],
    scratch_types=[
        pltpu.VMEM((KB, 128), jnp.int32),
        pltpu.VMEM((128, CNTW), jnp.float32),
        pltpu.VMEM_SHARED((ACC_N, CNTW), jnp.float32),
    ],
)(_cnt_body)


def _mm(a, b):
  return jnp.dot(a, b, preferred_element_type=jnp.float32,
                 precision=lax.Precision.HIGHEST)


def _proj_body(x_ref, w_ref, b_ref, o_ref):
  o_ref[...] = _mm(x_ref[...], w_ref[...]) + b_ref[...]


_proj = pl.pallas_call(
    _proj_body,
    grid=(50,),
    in_specs=[
        pl.BlockSpec((1000, D), lambda i: (i, 0)),
        pl.BlockSpec((D, H), lambda i: (0, 0)),
        pl.BlockSpec((1, H), lambda i: (0, 0)),
    ],
    out_specs=pl.BlockSpec((1000, H), lambda i: (i, 0)),
    out_shape=jax.ShapeDtypeStruct((N, H), jnp.float32),
)


def _comb_one(i, s_ref, c_ref, h_ref, wl_ref, wr_ref, bl_ref, o_ref, st_ref):
  sv = s_ref[...]
  t = _mm(sv[0], wl_ref[0])
  for k in range(1, NCHUNK):
    t += _mm(sv[k], wl_ref[k])
  cinv = 1.0 / jnp.maximum(c_ref[...][:, :1], 1.0)
  out = t * cinv + _mm(h_ref[...], wr_ref[...]) + bl_ref[...]
  o_ref[...] = out

  @pl.when(i == 0)
  def _():
    st_ref[...] = jnp.zeros_like(st_ref)

  st_ref[0:1, :] += jnp.sum(out, axis=0, keepdims=True)
  st_ref[1:2, :] += jnp.sum(out * out, axis=0, keepdims=True)


def _comb_body(s_ref, c_ref, h_ref, wl_ref, wr_ref, bl_ref, o_ref, st_ref):
  _comb_one(pl.program_id(0), s_ref, c_ref, h_ref, wl_ref, wr_ref, bl_ref,
            o_ref, st_ref)


_comb = pl.pallas_call(
    _comb_body,
    grid=(50,),
    in_specs=[
        pl.BlockSpec((NCHUNK, 1000, CW), lambda i: (0, i, 0)),
        pl.BlockSpec((1000, CNTW), lambda i: (i, 0)),
        pl.BlockSpec((1000, D), lambda i: (i, 0)),
        pl.BlockSpec((NCHUNK, CW, H), lambda i: (0, 0, 0)),
        pl.BlockSpec((D, H), lambda i: (0, 0)),
        pl.BlockSpec((1, H), lambda i: (0, 0)),
    ],
    out_specs=[
        pl.BlockSpec((1000, H), lambda i: (i, 0)),
        pl.BlockSpec((8, H), lambda i: (0, 0)),
    ],
    out_shape=[
        jax.ShapeDtypeStruct((N, H), jnp.float32),
        jax.ShapeDtypeStruct((8, H), jnp.float32),
    ],
)


def _bn_body(x_ref, sc_ref, sh_ref, o_ref):
  o_ref[...] = jnp.maximum(x_ref[...] * sc_ref[...] + sh_ref[...], 0.0)


_bn = pl.pallas_call(
    _bn_body,
    grid=(50,),
    in_specs=[
        pl.BlockSpec((1000, H), lambda i: (i, 0)),
        pl.BlockSpec((1, H), lambda i: (0, 0)),
        pl.BlockSpec((1, H), lambda i: (0, 0)),
    ],
    out_specs=pl.BlockSpec((1000, H), lambda i: (i, 0)),
    out_shape=jax.ShapeDtypeStruct((N, H), jnp.float32),
)


def _pad_edges(edge):
  npad = E_PAD - E
  ar = jnp.arange(npad, dtype=jnp.int32)
  pad_src = (ar * 37) % N
  pad_dst = N + ar % PAD_ROWS
  src2 = jnp.concatenate([edge[0], pad_src]).reshape(-1, 128)
  dst2 = jnp.concatenate([edge[1], pad_dst]).reshape(-1, 128)
  return src2, dst2


def _bn_coeffs(st, g, be):
  mu = st[0] / N
  var = jnp.maximum(st[1] / N - mu * mu, 0.0)
  scale = g / jnp.sqrt(var + EPS)
  shift = be - mu * scale
  return scale.reshape(1, H), shift.reshape(1, H)


def kernel(x_user, x_item, edge_ui, edge_iu,
           W_in_user, b_in_user, W_in_item, b_in_item,
           Wl0_ui, bl0_ui, Wr0_ui, Wl0_iu, bl0_iu, Wr0_iu,
           g0_user, be0_user, g0_item, be0_item,
           Wl1_ui, bl1_ui, Wr1_ui, Wl1_iu, bl1_iu, Wr1_iu,
           g1_user, be1_user, g1_item, be1_item):
  src_ui, dst_ui = _pad_edges(edge_ui)
  src_iu, dst_iu = _pad_edges(edge_iu)
  zeros32 = jnp.zeros((ZROWS, CW), jnp.float32)
  zeros8 = jnp.zeros((ZROWS, CNTW), jnp.float32)
  ones8 = jnp.ones((128, CNTW), jnp.float32)

  c_i, c_u = _cnt(dst_ui, dst_iu, zeros8, ones8)

  h_user = _proj(x_user, W_in_user, b_in_user.reshape(1, H))
  h_item = _proj(x_item, W_in_item, b_in_item.reshape(1, H))

  params = [
      (Wl0_ui, bl0_ui, Wr0_ui, Wl0_iu, bl0_iu, Wr0_iu,
       g0_user, be0_user, g0_item, be0_item),
      (Wl1_ui, bl1_ui, Wr1_ui, Wl1_iu, bl1_iu, Wr1_iu,
       g1_user, be1_user, g1_item, be1_item),
  ]
  for (Wl_ui, bl_ui, Wr_ui, Wl_iu, bl_iu, Wr_iu,
       g_u, be_u, g_i, be_i) in params:
    s_i = _seg(h_user.reshape(N * NCHUNK, CW), src_ui, dst_ui, zeros32)
    s_u = _seg(h_item.reshape(N * NCHUNK, CW), src_iu, dst_iu, zeros32)
    out_i, st_i = _comb(s_i, c_i, h_item, Wl_ui.reshape(NCHUNK, CW, H),
                        Wr_ui, bl_ui.reshape(1, H))
    out_u, st_u = _comb(s_u, c_u, h_user, Wl_iu.reshape(NCHUNK, CW, H),
                        Wr_iu, bl_iu.reshape(1, H))
    sc_u, sh_u = _bn_coeffs(st_u, g_u, be_u)
    sc_i, sh_i = _bn_coeffs(st_i, g_i, be_i)
    h_user = _bn(out_u, sc_u, sh_u)
    h_item = _bn(out_i, sc_i, sh_i)

  return (h_user, h_item)


# speedup vs baseline: 2.0059x; 1.2942x over previous
import functools

import jax
import jax.numpy as jnp
from jax import lax
from jax.experimental import pallas as pl
from jax.experimental.pallas import tpu as pltpu
from jax.experimental.pallas import tpu_sc as plsc

N = 50000
E = 625000
D = 128
H = 128
EPS = 1e-5

NCHUNK = 4
CW = 32
NTILE = 16
BLK = 1280
NBLK = 31
E_PAD = NTILE * NBLK * BLK
PAD_ROWS = 304
ACC_N = N + PAD_ROWS
ZROWS = ACC_N // NTILE
OROWS = 3128
CNTW = 8
KB = BLK // 128


def _seg_body(h4, src2, dst2, zeros32, s_out, gi2, dsw2, rows, acc_sh, sems):
  c = lax.axis_index("c")
  s = lax.axis_index("s")
  G = NBLK * KB

  for l in range(2):
    cc = 2 * c + l
    pltpu.sync_copy(zeros32, acc_sh.at[pl.ds(ZROWS * s, ZROWS)])
    plsc.subcore_barrier()

    def fire_idx(j, p):
      row0 = (s * NBLK + j) * KB
      pltpu.async_copy(src2.at[pl.ds(row0, KB)],
                       gi2.at[pl.ds(p * KB, KB)], sems.at[p])
      pltpu.async_copy(dst2.at[pl.ds(row0, KB)],
                       dsw2.at[pl.ds(p * KB, KB)], sems.at[p])

    def wait_idx(j, p):
      row0 = (s * NBLK + j) * KB
      pltpu.make_async_copy(src2.at[pl.ds(row0, KB)],
                            gi2.at[pl.ds(p * KB, KB)], sems.at[p]).wait()
      pltpu.make_async_copy(dst2.at[pl.ds(row0, KB)],
                            dsw2.at[pl.ds(p * KB, KB)], sems.at[p]).wait()

    def transform(p):
      def outer(k, _):
        def inner(i, _):
          gi2[p * KB + k, pl.ds(i * 16, 16)] = (
              gi2[p * KB + k, pl.ds(i * 16, 16)] * NCHUNK + cc)
          return None
        lax.fori_loop(0, 8, inner, None)
        return None
      lax.fori_loop(0, KB, outer, None)

    def fire_gather(t):
      p = (t // KB) % 2
      rb = t % 6
      pltpu.async_copy(h4.at[gi2.at[p * KB + t % KB]],
                       rows.at[pl.ds(rb * 128, 128)], sems.at[2 + rb])

    def wait_gather(t):
      rb = t % 6
      pltpu.make_async_copy(h4.at[pl.ds(0, 128)],
                            rows.at[pl.ds(rb * 128, 128)],
                            sems.at[2 + rb]).wait()

    def fire_scatter(t):
      p = (t // KB) % 2
      rb = t % 6
      pltpu.async_copy(rows.at[pl.ds(rb * 128, 128)],
                       acc_sh.at[dsw2.at[p * KB + t % KB]],
                       sems.at[8 + rb], add=True)

    def wait_scatter(t):
      rb = t % 6
      pltpu.make_async_copy(h4.at[pl.ds(0, 128)],
                            rows.at[pl.ds(rb * 128, 128)],
                            sems.at[8 + rb]).wait()

    fire_idx(0, 0)
    wait_idx(0, 0)
    transform(0)
    fire_idx(1, 1)
    fire_gather(0)
    fire_gather(1)
    fire_gather(2)
    fire_gather(3)
    fire_gather(4)

    def body(t, _):
      t3 = t + 5

      @pl.when((t3 % KB == 0) & (t3 < G))
      def _():
        wait_idx(t3 // KB, (t3 // KB) % 2)
        transform((t3 // KB) % 2)

      wait_gather(t)
      fire_scatter(t)

      @pl.when(t3 < G)
      def _():
        @pl.when(t >= 1)
        def _():
          wait_scatter(t3)

        fire_gather(t3)

      @pl.when((t % KB == 2) & (t // KB >= 1) & (t // KB + 1 < NBLK))
      def _():
        fire_idx(t // KB + 1, (t // KB + 1) % 2)
      return None

    lax.fori_loop(0, G, body, None)
    wait_scatter(G - 6)
    wait_scatter(G - 5)
    wait_scatter(G - 4)
    wait_scatter(G - 3)
    wait_scatter(G - 2)
    wait_scatter(G - 1)
    plsc.subcore_barrier()

    last = N - 15 * OROWS

    @pl.when(s < 15)
    def _():
      pltpu.sync_copy(acc_sh.at[pl.ds(OROWS * s, OROWS)],
                      s_out.at[cc, pl.ds(OROWS * s, OROWS)])

    @pl.when(s == 15)
    def _():
      pltpu.sync_copy(acc_sh.at[pl.ds(OROWS * 15, last)],
                      s_out.at[cc, pl.ds(OROWS * 15, last)])
    plsc.subcore_barrier()


_seg = functools.partial(
    pl.kernel,
    mesh=plsc.VectorSubcoreMesh(core_axis_name="c", subcore_axis_name="s"),
    compiler_params=pltpu.CompilerParams(use_tc_tiling_on_sc=False),
    out_type=jax.ShapeDtypeStruct((NCHUNK, N, CW), jnp.float32),
    scratch_types=[
        pltpu.VMEM((2 * KB, 128), jnp.int32),
        pltpu.VMEM((2 * KB, 128), jnp.int32),
        pltpu.VMEM((768, CW), jnp.float32),
        pltpu.VMEM_SHARED((ACC_N, CW), jnp.float32),
        pltpu.SemaphoreType.DMA((14,)),
    ],
)(_seg_body)


def _cnt_body(dst2_ui, dst2_iu, zeros8, ones8, cui_out, ciu_out,
              dstw, ones_v, cnt_sh):
  c = lax.axis_index("c")
  s = lax.axis_index("s")

  pltpu.sync_copy(ones8, ones_v)
  pltpu.sync_copy(zeros8, cnt_sh.at[pl.ds(ZROWS * s, ZROWS)])
  plsc.subcore_barrier()

  def count(dst2):
    def block(j, _):
      row0 = (s * NBLK + j) * KB
      pltpu.sync_copy(dst2.at[pl.ds(row0, KB)], dstw)
      for k in range(KB):
        pltpu.sync_copy(ones_v, cnt_sh.at[dstw.at[k]], add=True)
      return None
    lax.fori_loop(0, NBLK, block, None)

  def copy_out(c_out):
    last = N - 15 * OROWS

    @pl.when(s < 15)
    def _():
      pltpu.sync_copy(cnt_sh.at[pl.ds(OROWS * s, OROWS)],
                      c_out.at[pl.ds(OROWS * s, OROWS)])

    @pl.when(s == 15)
    def _():
      pltpu.sync_copy(cnt_sh.at[pl.ds(OROWS * 15, last)],
                      c_out.at[pl.ds(OROWS * 15, last)])

  @pl.when(c == 0)
  def _():
    count(dst2_ui)
  @pl.when(c == 1)
  def _():
    count(dst2_iu)
  plsc.subcore_barrier()

  @pl.when(c == 0)
  def _():
    copy_out(cui_out)
  @pl.when(c == 1)
  def _():
    copy_out(ciu_out)


_cnt = functools.partial(
    pl.kernel,
    mesh=plsc.VectorSubcoreMesh(core_axis_name="c", subcore_axis_name="s"),
    compiler_params=pltpu.CompilerParams(use_tc_tiling_on_sc=False),
    out_type=[
        jax.ShapeDtypeStruct((N, CNTW), jnp.float32),
        jax.ShapeDtypeStruct((N, CNTW), jnp.float32),
    ],
    scratch_types=[
        pltpu.VMEM((KB, 128), jnp.int32),
        pltpu.VMEM((128, CNTW), jnp.float32),
        pltpu.VMEM_SHARED((ACC_N, CNTW), jnp.float32),
    ],
)(_cnt_body)


def _mm(a, b):
  return jnp.dot(a, b, preferred_element_type=jnp.float32)


def _proj_body(x_ref, w_ref, b_ref, o_ref):
  o_ref[...] = _mm(x_ref[...], w_ref[...]) + b_ref[...]


_proj = pl.pallas_call(
    _proj_body,
    grid=(50,),
    in_specs=[
        pl.BlockSpec((1000, D), lambda i: (i, 0)),
        pl.BlockSpec((D, H), lambda i: (0, 0)),
        pl.BlockSpec((1, H), lambda i: (0, 0)),
    ],
    out_specs=pl.BlockSpec((1000, H), lambda i: (i, 0)),
    out_shape=jax.ShapeDtypeStruct((N, H), jnp.float32),
)


def _comb_one(i, s_ref, c_ref, h_ref, wl_ref, wr_ref, bl_ref, o_ref, st_ref):
  sv = s_ref[...]
  t = _mm(sv[0], wl_ref[0])
  for k in range(1, NCHUNK):
    t += _mm(sv[k], wl_ref[k])
  cinv = 1.0 / jnp.maximum(c_ref[...][:, :1], 1.0)
  out = t * cinv + _mm(h_ref[...], wr_ref[...]) + bl_ref[...]
  o_ref[...] = out

  @pl.when(i == 0)
  def _():
    st_ref[...] = jnp.zeros_like(st_ref)

  st_ref[0:1, :] += jnp.sum(out, axis=0, keepdims=True)
  st_ref[1:2, :] += jnp.sum(out * out, axis=0, keepdims=True)


def _comb_body(s_ref, c_ref, h_ref, wl_ref, wr_ref, bl_ref, o_ref, st_ref):
  _comb_one(pl.program_id(0), s_ref, c_ref, h_ref, wl_ref, wr_ref, bl_ref,
            o_ref, st_ref)


_comb = pl.pallas_call(
    _comb_body,
    grid=(50,),
    in_specs=[
        pl.BlockSpec((NCHUNK, 1000, CW), lambda i: (0, i, 0)),
        pl.BlockSpec((1000, CNTW), lambda i: (i, 0)),
        pl.BlockSpec((1000, D), lambda i: (i, 0)),
        pl.BlockSpec((NCHUNK, CW, H), lambda i: (0, 0, 0)),
        pl.BlockSpec((D, H), lambda i: (0, 0)),
        pl.BlockSpec((1, H), lambda i: (0, 0)),
    ],
    out_specs=[
        pl.BlockSpec((1000, H), lambda i: (i, 0)),
        pl.BlockSpec((8, H), lambda i: (0, 0)),
    ],
    out_shape=[
        jax.ShapeDtypeStruct((N, H), jnp.float32),
        jax.ShapeDtypeStruct((8, H), jnp.float32),
    ],
)


def _bn_body(x_ref, sc_ref, sh_ref, o_ref):
  o_ref[...] = jnp.maximum(x_ref[...] * sc_ref[...] + sh_ref[...], 0.0)


_bn = pl.pallas_call(
    _bn_body,
    grid=(50,),
    in_specs=[
        pl.BlockSpec((1000, H), lambda i: (i, 0)),
        pl.BlockSpec((1, H), lambda i: (0, 0)),
        pl.BlockSpec((1, H), lambda i: (0, 0)),
    ],
    out_specs=pl.BlockSpec((1000, H), lambda i: (i, 0)),
    out_shape=jax.ShapeDtypeStruct((N, H), jnp.float32),
)


def _pad_edges(edge):
  npad = E_PAD - E
  ar = jnp.arange(npad, dtype=jnp.int32)
  pad_src = (ar * 37) % N
  pad_dst = N + ar % PAD_ROWS
  src2 = jnp.concatenate([edge[0], pad_src]).reshape(-1, 128)
  dst2 = jnp.concatenate([edge[1], pad_dst]).reshape(-1, 128)
  return src2, dst2


def _bn_coeffs(st, g, be):
  mu = st[0] / N
  var = jnp.maximum(st[1] / N - mu * mu, 0.0)
  scale = g / jnp.sqrt(var + EPS)
  shift = be - mu * scale
  return scale.reshape(1, H), shift.reshape(1, H)


def kernel(x_user, x_item, edge_ui, edge_iu,
           W_in_user, b_in_user, W_in_item, b_in_item,
           Wl0_ui, bl0_ui, Wr0_ui, Wl0_iu, bl0_iu, Wr0_iu,
           g0_user, be0_user, g0_item, be0_item,
           Wl1_ui, bl1_ui, Wr1_ui, Wl1_iu, bl1_iu, Wr1_iu,
           g1_user, be1_user, g1_item, be1_item):
  src_ui, dst_ui = _pad_edges(edge_ui)
  src_iu, dst_iu = _pad_edges(edge_iu)
  zeros32 = jnp.zeros((ZROWS, CW), jnp.float32)
  zeros8 = jnp.zeros((ZROWS, CNTW), jnp.float32)
  ones8 = jnp.ones((128, CNTW), jnp.float32)

  c_i, c_u = _cnt(dst_ui, dst_iu, zeros8, ones8)

  h_user = _proj(x_user, W_in_user, b_in_user.reshape(1, H))
  h_item = _proj(x_item, W_in_item, b_in_item.reshape(1, H))

  params = [
      (Wl0_ui, bl0_ui, Wr0_ui, Wl0_iu, bl0_iu, Wr0_iu,
       g0_user, be0_user, g0_item, be0_item),
      (Wl1_ui, bl1_ui, Wr1_ui, Wl1_iu, bl1_iu, Wr1_iu,
       g1_user, be1_user, g1_item, be1_item),
  ]
  for (Wl_ui, bl_ui, Wr_ui, Wl_iu, bl_iu, Wr_iu,
       g_u, be_u, g_i, be_i) in params:
    s_i = _seg(h_user.reshape(N * NCHUNK, CW), src_ui, dst_ui, zeros32)
    s_u = _seg(h_item.reshape(N * NCHUNK, CW), src_iu, dst_iu, zeros32)
    out_i, st_i = _comb(s_i, c_i, h_item, Wl_ui.reshape(NCHUNK, CW, H),
                        Wr_ui, bl_ui.reshape(1, H))
    out_u, st_u = _comb(s_u, c_u, h_user, Wl_iu.reshape(NCHUNK, CW, H),
                        Wr_iu, bl_iu.reshape(1, H))
    sc_u, sh_u = _bn_coeffs(st_u, g_u, be_u)
    sc_i, sh_i = _bn_coeffs(st_i, g_i, be_i)
    h_user = _bn(out_u, sc_u, sh_u)
    h_item = _bn(out_i, sc_i, sh_i)

  return (h_user, h_item)
